# Initial kernel scaffold; baseline (speedup 1.0000x reference)
import jax
import jax.numpy as jnp
from jax.experimental import pallas as pl


def kernel(edge_index, features, edge_weights, W, b):
    raise NotImplementedError("write your pallas kernel here")


# speedup vs baseline: 3.3393x; 3.3393x over previous
import functools

import jax
import jax.numpy as jnp
from jax import lax
from jax.experimental import pallas as pl
from jax.experimental.pallas import tpu as pltpu
from jax.experimental.pallas import tpu_sc as plsc

NC = 2
NS = 16
NW = NC * NS
C = 128
L = 16

_SPLAT_DNUMS = lax.GatherDimensionNumbers(
    offset_dims=(), collapsed_slice_dims=(0,), start_index_map=(0,))


def _splat(v, l):
  idx = jnp.full((L, 1), l, jnp.int32)
  return lax.gather(v, idx, _SPLAT_DNUMS, slice_sizes=(1,),
                    mode=lax.GatherScatterMode.PROMISE_IN_BOUNDS)


def _propagate_kernel(n_nodes, n_chunks,
                      src_hbm, dst_hbm, w_hbm, feat_hbm, zeros_hbm, out_hbm,
                      src_v, dst_v, w_v, rows_v, acc_sh, sem):
  cid = lax.axis_index("c")
  sid = lax.axis_index("s")
  wid = cid * NS + sid

  @pl.when(sid == 0)
  def _():
    pltpu.sync_copy(zeros_hbm, acc_sh)

  base = wid * n_chunks
  pltpu.sync_copy(src_hbm.at[pl.ds(base, n_chunks)], src_v)
  pltpu.sync_copy(dst_hbm.at[pl.ds(base, n_chunks)], dst_v)
  pltpu.sync_copy(w_hbm.at[pl.ds(base, n_chunks)], w_v)

  plsc.subcore_barrier()

  def chunk_body(g, carry):
    pltpu.async_copy(feat_hbm.at[src_v.at[g]], rows_v, sem).wait()

    def group_body(k, carry2):
      wv = w_v[g, pl.ds(k * L, L)]
      for l in range(L):
        wb = _splat(wv, l)
        r = k * L + l
        for m in range(8):
          rows_v[r, pl.ds(m * L, L)] = rows_v[r, pl.ds(m * L, L)] * wb
      return carry2

    lax.fori_loop(0, C // L, group_body, 0)

    pltpu.sync_copy(rows_v, acc_sh.at[dst_v.at[g]], add=True)
    return carry

  lax.fori_loop(0, n_chunks, chunk_body, 0)

  plsc.subcore_barrier()

  @pl.when(sid == 0)
  def _():
    pltpu.sync_copy(acc_sh, out_hbm.at[cid])


def _finish_kernel(p0_ref, p1_ref, w_ref, b_ref, o_ref):
  a = p0_ref[...] + p1_ref[...]
  z = jnp.dot(a, w_ref[...], preferred_element_type=jnp.float32) + b_ref[...]
  m = jnp.max(z, axis=1, keepdims=True)
  zs = z - m
  s = jnp.sum(jnp.exp(zs), axis=1, keepdims=True)
  o_ref[...] = zs - jnp.log(s)


def kernel(edge_index, features, edge_weights, W, b):
  n = features.shape[0]
  e = edge_index.shape[1]
  d_in = features.shape[1]
  d_out = W.shape[1]

  per_w = -(-e // NW)
  n_chunks = -(-(-(-per_w // C)) // 8) * 8
  e_pad = NW * n_chunks * C
  pad = e_pad - e
  src = jnp.concatenate([edge_index[0], jnp.zeros((pad,), jnp.int32)])
  dst = jnp.concatenate([edge_index[1], jnp.zeros((pad,), jnp.int32)])
  ew = jnp.concatenate([edge_weights, jnp.zeros((pad,), jnp.float32)])
  src2d = src.reshape(NW * n_chunks, C)
  dst2d = dst.reshape(NW * n_chunks, C)
  ew2d = ew.reshape(NW * n_chunks, C)
  zeros = jnp.zeros((n, d_in), jnp.float32)

  mesh = plsc.VectorSubcoreMesh(core_axis_name="c", subcore_axis_name="s")
  propagate = pl.kernel(
      functools.partial(_propagate_kernel, n, n_chunks),
      out_type=jax.ShapeDtypeStruct((NC, n, d_in), jnp.float32),
      mesh=mesh,
      scratch_types=[
          pltpu.VMEM((n_chunks, C), jnp.int32),
          pltpu.VMEM((n_chunks, C), jnp.int32),
          pltpu.VMEM((n_chunks, C), jnp.float32),
          pltpu.VMEM((C, d_in), jnp.float32),
          pltpu.VMEM_SHARED((n, d_in), jnp.float32),
          pltpu.SemaphoreType.DMA,
      ],
  )
  partials = propagate(src2d, dst2d, ew2d, features, zeros)

  bn = 1000
  out = pl.pallas_call(
      _finish_kernel,
      grid=(n // bn,),
      in_specs=[
          pl.BlockSpec((bn, d_in), lambda i: (i, 0)),
          pl.BlockSpec((bn, d_in), lambda i: (i, 0)),
          pl.BlockSpec((d_in, d_out), lambda i: (0, 0)),
          pl.BlockSpec((1, d_out), lambda i: (0, 0)),
      ],
      out_specs=pl.BlockSpec((bn, d_out), lambda i: (i, 0)),
      out_shape=jax.ShapeDtypeStruct((n, d_out), jnp.float32),
  )(partials[0], partials[1], W, b.reshape(1, d_out))
  return out
